# baseline (speedup 1.0000x reference)
import functools

import jax
import jax.numpy as jnp
from jax import lax
from jax.experimental import pallas as pl
from jax.experimental.pallas import tpu as pltpu
from jax.experimental.pallas import tpu_sc as plsc

DIM = 64
WIDE = 128
LANES = 16


@functools.cache
def _make_gather(b: int, l: int, dim: int):
    info = plsc.get_sparse_core_info()
    nw = info.num_cores * info.num_subcores
    assert b % nw == 0
    bw = b // nw
    assert bw == 128 and l % 2 == 0
    mesh = plsc.VectorSubcoreMesh(core_axis_name="c", subcore_axis_name="s")

    @functools.partial(
        pl.kernel,
        mesh=mesh,
        out_type=jax.ShapeDtypeStruct((l, dim, b), jnp.float32),
        scratch_types=[
            pltpu.VMEM((bw * l + LANES,), jnp.int32),
            pltpu.VMEM((l * bw,), jnp.int32),
            pltpu.VMEM((2, bw, WIDE), jnp.float32),
            pltpu.VMEM((2, dim, bw), jnp.float32),
            pltpu.SemaphoreType.DMA,
            pltpu.SemaphoreType.DMA,
        ],
        compiler_params=pltpu.CompilerParams(
            use_tc_tiling_on_sc=True, needs_layout_passes=False
        ),
    )
    def gather_kernel(tok_hbm, table_hbm, out_hbm, idx_v, idxt_v, rows_v, tbuf_v, gsem, osem):
        wid = lax.axis_index("s") * info.num_cores + lax.axis_index("c")
        b0 = wid * bw
        pltpu.sync_copy(
            tok_hbm.at[pl.ds(b0 * l, bw * l)], idx_v.at[pl.ds(0, bw * l)]
        )

        iota = lax.iota(jnp.int32, LANES)
        nfull = l // LANES
        rem = l % LANES

        @pl.loop(0, bw)
        def _tr_idx(bp):
            for q in range(nfull):
                vals = idx_v[pl.ds(bp * l + q * LANES, LANES)]
                pos = (iota + q * LANES) * bw + bp
                plsc.store_scatter(idxt_v, [pos], vals)
            if rem:
                vals = idx_v[pl.ds(bp * l + nfull * LANES, LANES)]
                pos = (iota + nfull * LANES) * bw + bp
                plsc.store_scatter(idxt_v, [pos], vals, mask=iota < rem)

        def issue_gather(lq, slot):
            pltpu.async_copy(
                table_hbm.at[idxt_v.at[pl.ds(lq * bw, bw)]],
                rows_v.at[slot],
                gsem,
            )

        def wait_gather(slot):
            pltpu.make_async_copy(
                table_hbm.at[idxt_v.at[pl.ds(0, bw)]],
                rows_v.at[slot],
                gsem,
            ).wait()

        def issue_write(lq, slot):
            pltpu.async_copy(
                tbuf_v.at[slot],
                out_hbm.at[lq, :, pl.ds(b0, bw)],
                osem,
            )

        def wait_write(slot):
            pltpu.make_async_copy(
                tbuf_v.at[slot],
                out_hbm.at[0, :, pl.ds(b0, bw)],
                osem,
            ).wait()

        issue_gather(0, 0)

        @pl.loop(0, l, step=2)
        def outer(l0):
            for s in range(2):
                lq = l0 + s
                wait_gather(s)

                @pl.when(lq + 1 < l)
                def _issue():
                    issue_gather(lq + 1, 1 - s)

                @pl.when(lq >= 2)
                def _drain():
                    wait_write(s)

                @pl.loop(0, (dim // LANES) * (bw // LANES), unroll=8)
                def _tr(blk):
                    kk = blk // (bw // LANES) * LANES
                    bb = blk % (bw // LANES) * LANES
                    brow = bb + iota
                    for r in range(LANES):
                        drow = kk + ((iota + r) & (LANES - 1))
                        vals = plsc.load_gather(rows_v.at[s], [brow, drow])
                        plsc.store_scatter(tbuf_v.at[s], [drow, brow], vals)

                issue_write(lq, s)

        for s in range(2):
            wait_write(s)

    return gather_kernel


def kernel(tok, table):
    b, l = tok.shape
    gather_kernel = _make_gather(b, l, DIM)
    table_wide = jnp.pad(table, ((0, 0), (0, WIDE - DIM)))
    out_ldb = gather_kernel(tok.reshape(-1), table_wide)
    return jnp.transpose(out_ldb, (2, 0, 1))


# speedup vs baseline: 1.1400x; 1.1400x over previous
import functools

import jax
import jax.numpy as jnp
from jax import lax
from jax.experimental import pallas as pl
from jax.experimental.pallas import tpu as pltpu
from jax.experimental.pallas import tpu_sc as plsc

DIM = 64
WIDE = 128
LANES = 16


@functools.cache
def _make_gather(b: int, l: int, dim: int):
    info = plsc.get_sparse_core_info()
    nw = info.num_cores * info.num_subcores
    assert b % nw == 0
    bw = b // nw
    assert bw == 128 and l % 2 == 0
    mesh = plsc.VectorSubcoreMesh(core_axis_name="c", subcore_axis_name="s")

    @functools.partial(
        pl.kernel,
        mesh=mesh,
        out_type=jax.ShapeDtypeStruct((l, dim, b), jnp.float32),
        scratch_types=[
            pltpu.VMEM((bw * l + LANES,), jnp.int32),
            pltpu.VMEM((l * bw,), jnp.int32),
            pltpu.VMEM((2, bw, WIDE), jnp.float32),
            pltpu.VMEM((2, dim, bw), jnp.float32),
            pltpu.SemaphoreType.DMA,
            pltpu.SemaphoreType.DMA,
        ],
        compiler_params=pltpu.CompilerParams(
            use_tc_tiling_on_sc=True, needs_layout_passes=False
        ),
    )
    def gather_kernel(tok_hbm, table_hbm, out_hbm, idx_v, idxt_v, rows_v, tbuf_v, gsem, osem):
        wid = lax.axis_index("s") * info.num_cores + lax.axis_index("c")
        b0 = wid * bw
        pltpu.sync_copy(
            tok_hbm.at[pl.ds(b0 * l, bw * l)], idx_v.at[pl.ds(0, bw * l)]
        )

        iota = lax.iota(jnp.int32, LANES)
        nfull = l // LANES
        rem = l % LANES

        @pl.loop(0, bw)
        def _tr_idx(bp):
            for q in range(nfull):
                vals = idx_v[pl.ds(bp * l + q * LANES, LANES)]
                pos = (iota + q * LANES) * bw + bp
                plsc.store_scatter(idxt_v, [pos], vals)
            if rem:
                vals = idx_v[pl.ds(bp * l + nfull * LANES, LANES)]
                pos = (iota + nfull * LANES) * bw + bp
                plsc.store_scatter(idxt_v, [pos], vals, mask=iota < rem)

        def issue_gather(lq, slot):
            pltpu.async_copy(
                table_hbm.at[idxt_v.at[pl.ds(lq * bw, bw)]],
                rows_v.at[slot],
                gsem,
            )

        def wait_gather(slot):
            pltpu.make_async_copy(
                table_hbm.at[idxt_v.at[pl.ds(0, bw)]],
                rows_v.at[slot],
                gsem,
            ).wait()

        def issue_write(lq, slot):
            pltpu.async_copy(
                tbuf_v.at[slot],
                out_hbm.at[lq, :, pl.ds(b0, bw)],
                osem,
            )

        def wait_write(slot):
            pltpu.make_async_copy(
                tbuf_v.at[slot],
                out_hbm.at[0, :, pl.ds(b0, bw)],
                osem,
            ).wait()

        issue_gather(0, 0)

        @pl.loop(0, l, step=2)
        def outer(l0):
            for s in range(2):
                lq = l0 + s
                wait_gather(s)

                @pl.when(lq + 1 < l)
                def _issue():
                    issue_gather(lq + 1, 1 - s)

                @pl.when(lq >= 2)
                def _drain():
                    wait_write(s)

                @pl.loop(0, (dim // LANES) * (bw // LANES), unroll=4)
                def _tr(blk):
                    kk = blk // (bw // LANES) * LANES
                    bb = blk % (bw // LANES) * LANES
                    brow = bb + iota
                    for r in range(LANES):
                        drow = kk + ((iota + r) & (LANES - 1))
                        vals = plsc.load_gather(rows_v.at[s], [brow, drow])
                        plsc.store_scatter(tbuf_v.at[s], [drow, brow], vals)

                issue_write(lq, s)

        for s in range(2):
            wait_write(s)

    return gather_kernel


def kernel(tok, table):
    b, l = tok.shape
    gather_kernel = _make_gather(b, l, DIM)
    table_wide = jnp.pad(table, ((0, 0), (0, WIDE - DIM)))
    out_ldb = gather_kernel(tok.reshape(-1), table_wide)
    return jnp.transpose(out_ldb, (2, 0, 1))
